# baseline (speedup 1.0000x reference)
import functools

import jax
import jax.numpy as jnp
from jax import lax
from jax.experimental import pallas as pl
from jax.experimental.pallas import tpu as pltpu
from jax.experimental.pallas import tpu_sc as plsc

K = 8192
KC = 4096
D = 32
N = 16 * 1024
BT = 512
G = N // BT
COMMITMENT_COST = 0.25

NC = 2
NS = 16
NW = NC * NS
ROWS_PER_W = N // NW
CHUNK = 128
NCHUNK = ROWS_PER_W // CHUNK


def _bf16_rne(v):
    bits = lax.bitcast_convert_type(v, jnp.int32)
    r = bits + jnp.int32(0x7FFF) + lax.bitwise_and(
        lax.shift_right_logical(bits, 16), jnp.int32(1))
    r = lax.bitwise_and(r, jnp.int32(-65536))
    return lax.bitcast_convert_type(r, jnp.float32)


TW = 512


def _tc_body(x_ref, cbt_ref, a_ref, b_ref, iota_ref, idx_ref, loss_ref,
             d_scr):
    x = x_ref[...]
    a = a_ref[...]
    acc_r = None
    acc_t = None
    acc_i = None
    for c in range(K // KC):
        mv = None
        for t in range(KC // TW):
            col = c * KC + t * TW
            m2 = jnp.dot(x, cbt_ref[:, pl.ds(col, TW)],
                         preferred_element_type=jnp.float32)
            dt = (a + b_ref[:, pl.ds(col, TW)]) - m2
            d_scr[:, pl.ds(t * TW, TW)] = dt
            tmin = jnp.min(dt, axis=1, keepdims=True)
            mv = tmin if mv is None else jnp.minimum(mv, tmin)
        mi = None
        for t in range(KC // TW):
            dt = d_scr[:, pl.ds(t * TW, TW)]
            ii = iota_ref[:, pl.ds(c * KC + t * TW, TW)]
            mit = jnp.min(jnp.where(dt == mv, ii, 65536.0), axis=1)
            mi = mit if mi is None else jnp.minimum(mi, mit)
        mv = mv[:, 0]
        if acc_i is None:
            acc_r, acc_t, acc_i = mv, mv, mi
        else:
            lt = mv < acc_r
            eq = (mv == acc_r) & (mi < acc_i)
            acc_i = jnp.where(lt | eq, mi, acc_i)
            acc_t = jnp.where(lt, mv, acc_t)
            acc_r = jnp.where(lt, mv, acc_r)
        acc_r = _bf16_rne(acc_r)
    idx_ref[0, 0, :] = acc_i.astype(jnp.int32)
    loss_ref[0, 0, :] = jnp.sum(acc_t).reshape(1)


_tc_quantize = pl.pallas_call(
    _tc_body,
    grid=(G,),
    in_specs=[
        pl.BlockSpec((BT, D), lambda i: (i, 0)),
        pl.BlockSpec((D, K), lambda i: (0, 0)),
        pl.BlockSpec((BT, 1), lambda i: (i, 0)),
        pl.BlockSpec((1, K), lambda i: (0, 0)),
        pl.BlockSpec((1, K), lambda i: (0, 0)),
    ],
    out_specs=[
        pl.BlockSpec((1, 1, BT), lambda i: (i, 0, 0)),
        pl.BlockSpec((1, 1, 1), lambda i: (i, 0, 0)),
    ],
    out_shape=[
        jax.ShapeDtypeStruct((G, 1, BT), jnp.int32),
        jax.ShapeDtypeStruct((G, 1, 1), jnp.float32),
    ],
    scratch_shapes=[pltpu.VMEM((BT, KC), jnp.float32)],
)


def _sc_gather(codebook, idx2):
    mesh = plsc.VectorSubcoreMesh(core_axis_name="c", subcore_axis_name="s")

    @functools.partial(
        pl.kernel,
        mesh=mesh,
        compiler_params=pltpu.CompilerParams(use_tc_tiling_on_sc=False),
        out_type=jax.ShapeDtypeStruct((N // CHUNK, CHUNK, D), jnp.float32),
        scratch_types=[
            pltpu.VMEM((NCHUNK, CHUNK), jnp.int32),
            pltpu.VMEM((NCHUNK, CHUNK, D), jnp.float32),
            pltpu.SemaphoreType.DMA,
        ],
    )
    def gather_kernel(cb_hbm, idx_hbm, out_hbm, idx_v, rows_v, sem):
        wid = lax.axis_index("s") * NC + lax.axis_index("c")
        base = wid * NCHUNK
        pltpu.sync_copy(idx_hbm.at[pl.ds(base, NCHUNK)], idx_v)
        for j in range(NCHUNK):
            pltpu.async_copy(cb_hbm.at[idx_v.at[j]], rows_v.at[j], sem).wait()
        pltpu.sync_copy(rows_v, out_hbm.at[pl.ds(base, NCHUNK)])

    return gather_kernel(codebook, idx2)


def kernel(inputs, codebook):
    flat = inputs.reshape(-1, D)
    a = jnp.sum(flat ** 2, axis=1, keepdims=True)
    b = jnp.sum(codebook ** 2, axis=1)
    iota_row = jnp.arange(K, dtype=jnp.float32).reshape(1, K)
    idx3, loss_part = _tc_quantize(
        flat * 2.0, codebook.T, a, b.reshape(1, K), iota_row)
    idx_flat = idx3.reshape(N)
    quantized = _sc_gather(codebook, idx_flat.reshape(N // CHUNK, CHUNK))
    e_latent = jnp.sum(loss_part) / (N * D)
    loss = e_latent + COMMITMENT_COST * e_latent
    return (
        quantized.reshape(inputs.shape),
        loss,
        idx_flat.reshape(inputs.shape[:-1]),
    )


# speedup vs baseline: 1.0564x; 1.0564x over previous
import functools

import jax
import jax.numpy as jnp
from jax import lax
from jax.experimental import pallas as pl
from jax.experimental.pallas import tpu as pltpu
from jax.experimental.pallas import tpu_sc as plsc

K = 8192
KC = 4096
D = 32
N = 16 * 1024
BT = 512
G = N // BT
COMMITMENT_COST = 0.25

NC = 2
NS = 16
NW = NC * NS
ROWS_PER_W = N // NW
CHUNK = 128
NCHUNK = ROWS_PER_W // CHUNK


def _bf16_rne(v):
    bits = lax.bitcast_convert_type(v, jnp.int32)
    r = bits + jnp.int32(0x7FFF) + lax.bitwise_and(
        lax.shift_right_logical(bits, 16), jnp.int32(1))
    r = lax.bitwise_and(r, jnp.int32(-65536))
    return lax.bitcast_convert_type(r, jnp.float32)


TW = 512


def _tc_body(x_ref, cbt_ref, a_ref, b_ref, iota_ref, idx_ref, loss_ref):
    x = x_ref[...]
    a = a_ref[...]
    acc_r = None
    acc_t = None
    acc_i = None
    for c in range(K // KC):
        mv = None
        mi = None
        for t in range(KC // TW):
            col = c * KC + t * TW
            m2 = jnp.dot(x, cbt_ref[:, pl.ds(col, TW)],
                         preferred_element_type=jnp.float32)
            dt = (a + b_ref[:, pl.ds(col, TW)]) - m2
            tmin = jnp.min(dt, axis=1, keepdims=True)
            ii = iota_ref[:, pl.ds(col, TW)]
            tidx = jnp.min(jnp.where(dt == tmin, ii, 65536.0),
                           axis=1, keepdims=True)
            if mv is None:
                mv, mi = tmin, tidx
            else:
                lt2 = tmin < mv
                eq2 = (tmin == mv) & (tidx < mi)
                mi = jnp.where(lt2 | eq2, tidx, mi)
                mv = jnp.minimum(tmin, mv)
        mv = mv[:, 0]
        mi = mi[:, 0]
        if acc_i is None:
            acc_r, acc_t, acc_i = mv, mv, mi
        else:
            lt = mv < acc_r
            eq = (mv == acc_r) & (mi < acc_i)
            acc_i = jnp.where(lt | eq, mi, acc_i)
            acc_t = jnp.where(lt, mv, acc_t)
            acc_r = jnp.where(lt, mv, acc_r)
        acc_r = _bf16_rne(acc_r)
    idx_ref[0, 0, :] = acc_i.astype(jnp.int32)
    loss_ref[0, 0, :] = jnp.sum(acc_t).reshape(1)


_tc_quantize = pl.pallas_call(
    _tc_body,
    grid=(G,),
    in_specs=[
        pl.BlockSpec((BT, D), lambda i: (i, 0)),
        pl.BlockSpec((D, K), lambda i: (0, 0)),
        pl.BlockSpec((BT, 1), lambda i: (i, 0)),
        pl.BlockSpec((1, K), lambda i: (0, 0)),
        pl.BlockSpec((1, K), lambda i: (0, 0)),
    ],
    out_specs=[
        pl.BlockSpec((1, 1, BT), lambda i: (i, 0, 0)),
        pl.BlockSpec((1, 1, 1), lambda i: (i, 0, 0)),
    ],
    out_shape=[
        jax.ShapeDtypeStruct((G, 1, BT), jnp.int32),
        jax.ShapeDtypeStruct((G, 1, 1), jnp.float32),
    ],
)


def _sc_gather(codebook, idx2):
    mesh = plsc.VectorSubcoreMesh(core_axis_name="c", subcore_axis_name="s")

    @functools.partial(
        pl.kernel,
        mesh=mesh,
        compiler_params=pltpu.CompilerParams(use_tc_tiling_on_sc=False),
        out_type=jax.ShapeDtypeStruct((N // CHUNK, CHUNK, D), jnp.float32),
        scratch_types=[
            pltpu.VMEM((NCHUNK, CHUNK), jnp.int32),
            pltpu.VMEM((NCHUNK, CHUNK, D), jnp.float32),
            pltpu.SemaphoreType.DMA,
        ],
    )
    def gather_kernel(cb_hbm, idx_hbm, out_hbm, idx_v, rows_v, sem):
        wid = lax.axis_index("s") * NC + lax.axis_index("c")
        base = wid * NCHUNK
        pltpu.sync_copy(idx_hbm.at[pl.ds(base, NCHUNK)], idx_v)
        for j in range(NCHUNK):
            pltpu.async_copy(cb_hbm.at[idx_v.at[j]], rows_v.at[j], sem).wait()
        pltpu.sync_copy(rows_v, out_hbm.at[pl.ds(base, NCHUNK)])

    return gather_kernel(codebook, idx2)


def kernel(inputs, codebook):
    flat = inputs.reshape(-1, D)
    a = jnp.sum(flat ** 2, axis=1, keepdims=True)
    b = jnp.sum(codebook ** 2, axis=1)
    iota_row = jnp.arange(K, dtype=jnp.float32).reshape(1, K)
    idx3, loss_part = _tc_quantize(
        flat * 2.0, codebook.T, a, b.reshape(1, K), iota_row)
    idx_flat = idx3.reshape(N)
    quantized = _sc_gather(codebook, idx_flat.reshape(N // CHUNK, CHUNK))
    e_latent = jnp.sum(loss_part) / (N * D)
    loss = e_latent + COMMITMENT_COST * e_latent
    return (
        quantized.reshape(inputs.shape),
        loss,
        idx_flat.reshape(inputs.shape[:-1]),
    )
